# Initial kernel scaffold; baseline (speedup 1.0000x reference)
import jax
import jax.numpy as jnp
from jax import lax
from jax.experimental import pallas as pl
from jax.experimental.pallas import tpu as pltpu
from jax.experimental.pallas import tpu_sc as plsc

N = 10000
E = 320000
D = 128
NSPECIES = 8
CUTOFF = 5.0
NLAYERS = 2

NW = 32
NSUB = 16
EPW = E // NW
CHUNK = 80
NCHUNK = EPW // CHUNK
ROWS_PER_SUB = N // NSUB

_mesh = lambda: plsc.VectorSubcoreMesh(core_axis_name="c", subcore_axis_name="s")


def _edge_prep_body(pos_hbm, src_hbm, dst_hbm, d2_hbm, pos_v, src_v, dst_v, d2_v):
    cidx = lax.axis_index("c")
    sidx = lax.axis_index("s")
    wid = cidx * NSUB + sidx
    pltpu.sync_copy(pos_hbm, pos_v)
    pltpu.sync_copy(src_hbm.at[wid], src_v)
    pltpu.sync_copy(dst_hbm.at[wid], dst_v)

    def chunk(c, carry):
        for k in range(CHUNK // 16):
            sl = pl.ds(k * 16, 16)
            s = src_v[c, sl]
            d = dst_v[c, sl]
            s3 = s * 3
            d3 = d * 3
            dx = plsc.load_gather(pos_v, [d3]) - plsc.load_gather(pos_v, [s3])
            dy = plsc.load_gather(pos_v, [d3 + 1]) - plsc.load_gather(pos_v, [s3 + 1])
            dz = plsc.load_gather(pos_v, [d3 + 2]) - plsc.load_gather(pos_v, [s3 + 2])
            d2_v[c, sl] = dx * dx + dy * dy + dz * dz
        return carry

    lax.fori_loop(0, NCHUNK, chunk, 0)
    pltpu.sync_copy(d2_v, d2_hbm.at[wid])


def _make_edge_prep(interpret=False):
    return pl.kernel(
        _edge_prep_body,
        out_type=jax.ShapeDtypeStruct((NW, NCHUNK, CHUNK), jnp.float32),
        mesh=_mesh(),
        scratch_types=[
            pltpu.VMEM((N * 3,), jnp.float32),
            pltpu.VMEM((NCHUNK, CHUNK), jnp.int32),
            pltpu.VMEM((NCHUNK, CHUNK), jnp.int32),
            pltpu.VMEM((NCHUNK, CHUNK), jnp.float32),
        ],
        interpret=interpret,
    )


def _fcut_x0_body(d2_ref, sp_ref, emb_ref, fcut_ref, x0_ref):
    d2 = d2_ref[...]
    dist = jnp.sqrt(d2 + 1e-9)
    r = jnp.clip(dist / CUTOFF, 0.0, 1.0)
    fcut_ref[...] = 0.5 * (jnp.cos(jnp.pi * r) + 1.0)
    sp = sp_ref[...]
    onehot = (sp == lax.broadcasted_iota(jnp.int32, (1, NSPECIES), 1)).astype(jnp.float32)
    x0_ref[...] = jnp.dot(onehot, emb_ref[...], preferred_element_type=jnp.float32)


def _make_fcut_x0(interpret=False):
    return pl.pallas_call(
        _fcut_x0_body,
        out_shape=[
            jax.ShapeDtypeStruct((E // D, D), jnp.float32),
            jax.ShapeDtypeStruct((N, D), jnp.float32),
        ],
        interpret=interpret,
    )


def _spmm_body(x_hbm, src_hbm, dst_hbm, fcut_hbm, out_hbm,
               src_v, dst_v, fcut_v, rows_v, zero_v, agg_sp, sem):
    cidx = lax.axis_index("c")
    sidx = lax.axis_index("s")
    wid = cidx * NSUB + sidx

    pltpu.sync_copy(src_hbm.at[wid], src_v)
    pltpu.sync_copy(dst_hbm.at[wid], dst_v)
    pltpu.sync_copy(fcut_hbm.at[wid], fcut_v)

    def zrow(r, carry):
        for k in range(D // 16):
            zero_v[r, pl.ds(k * 16, 16)] = jnp.zeros((16,), jnp.float32)
        return carry

    lax.fori_loop(0, NCHUNK, zrow, 0)
    for i in range(ROWS_PER_SUB // NCHUNK):
        pltpu.sync_copy(zero_v, agg_sp.at[pl.ds(sidx * ROWS_PER_SUB + i * NCHUNK, NCHUNK)])
    plsc.subcore_barrier()

    def chunk(j, carry):
        pltpu.async_copy(x_hbm.at[src_v.at[j]], rows_v, sem).wait()
        js = jnp.full((16,), j, dtype=jnp.int32)
        for r in range(CHUNK):
            f = plsc.load_gather(fcut_v, [js, jnp.full((16,), r, dtype=jnp.int32)])
            for k in range(D // 16):
                sl = pl.ds(k * 16, 16)
                rows_v[r, sl] = rows_v[r, sl] * f
        pltpu.sync_copy(rows_v, agg_sp.at[dst_v.at[j]], add=True)
        return carry

    lax.fori_loop(0, NCHUNK, chunk, 0)
    plsc.subcore_barrier()

    pltpu.sync_copy(agg_sp.at[pl.ds(sidx * ROWS_PER_SUB, ROWS_PER_SUB)],
                    out_hbm.at[cidx, pl.ds(sidx * ROWS_PER_SUB, ROWS_PER_SUB)])


def _make_spmm(interpret=False):
    return pl.kernel(
        _spmm_body,
        out_type=jax.ShapeDtypeStruct((2, N, D), jnp.float32),
        mesh=_mesh(),
        scratch_types=[
            pltpu.VMEM((NCHUNK, CHUNK), jnp.int32),
            pltpu.VMEM((NCHUNK, CHUNK), jnp.int32),
            pltpu.VMEM((NCHUNK, CHUNK), jnp.float32),
            pltpu.VMEM((CHUNK, D), jnp.float32),
            pltpu.VMEM((NCHUNK, D), jnp.float32),
            pltpu.VMEM_SHARED((N, D), jnp.float32),
            pltpu.SemaphoreType.DMA,
        ],
        interpret=interpret,
    )


def _mix_body(x_ref, p0_ref, p1_ref, w_ref, out_ref):
    agg = p0_ref[...] + p1_ref[...]
    x = x_ref[...]
    t = jnp.tanh(jnp.dot(agg, w_ref[...], preferred_element_type=jnp.float32))
    out_ref[...] = x + t * x


def _make_mix(interpret=False):
    return pl.pallas_call(
        _mix_body,
        out_shape=jax.ShapeDtypeStruct((N, D), jnp.float32),
        interpret=interpret,
    )


def _final_body(x_ref, p0_ref, p1_ref, w_ref, wout_ref, sp_ref, comp_ref, out_ref):
    agg = p0_ref[...] + p1_ref[...]
    x = x_ref[...]
    t = jnp.tanh(jnp.dot(agg, w_ref[...], preferred_element_type=jnp.float32))
    x2 = x + t * x
    e = jnp.dot(x2, wout_ref[...], preferred_element_type=jnp.float32)
    onehot = (sp_ref[...] == lax.broadcasted_iota(jnp.int32, (1, NSPECIES), 1))
    comp_e = jnp.sum(jnp.where(onehot, comp_ref[...], 0.0))
    out_ref[0, 0] = jnp.sum(e) + comp_e


def _make_final(interpret=False):
    return pl.pallas_call(
        _final_body,
        out_shape=jax.ShapeDtypeStruct((1, 1), jnp.float32),
        interpret=interpret,
    )


def _run(positions, species, edge_index, embeddings, W_msg, W_out,
         composition_weights, interpret=False):
    src = edge_index[0].astype(jnp.int32).reshape(NW, NCHUNK, CHUNK)
    dst = edge_index[1].astype(jnp.int32).reshape(NW, NCHUNK, CHUNK)
    sp32 = species.astype(jnp.int32).reshape(N, 1)
    pos_flat = positions.astype(jnp.float32).reshape(N * 3)

    d2 = _make_edge_prep(interpret)(pos_flat, src, dst)
    fcut2d, x0 = _make_fcut_x0(interpret)(d2.reshape(E // D, D), sp32, embeddings)
    fcut = fcut2d.reshape(NW, NCHUNK, CHUNK)

    spmm = _make_spmm(interpret)
    mix = _make_mix(interpret)

    x = x0
    for l in range(NLAYERS - 1):
        partials = spmm(x, src, dst, fcut)
        x = mix(x, partials[0], partials[1], W_msg[l])
    partials = spmm(x, src, dst, fcut)
    out = _make_final(interpret)(x, partials[0], partials[1],
                                 W_msg[NLAYERS - 1], W_out, sp32,
                                 composition_weights.reshape(1, NSPECIES))
    return out.reshape(())


def kernel(positions, species, edge_index, embeddings, W_msg, W_out,
           composition_weights):
    return _run(positions, species, edge_index, embeddings, W_msg, W_out,
                composition_weights)


# speedup vs baseline: 7.6662x; 7.6662x over previous
import jax
import jax.numpy as jnp
from jax import lax
from jax.experimental import pallas as pl
from jax.experimental.pallas import tpu as pltpu
from jax.experimental.pallas import tpu_sc as plsc

N = 10000
E = 320000
D = 128
NSPECIES = 8
CUTOFF = 5.0
NLAYERS = 2

NW = 32
NSUB = 16
EPW = E // NW
CHUNK = 80
NCHUNK = EPW // CHUNK
ROWS_PER_SUB = N // NSUB

_mesh = lambda: plsc.VectorSubcoreMesh(core_axis_name="c", subcore_axis_name="s",
                                       num_cores=2, num_subcores=16)


def _edge_prep_body(pos_hbm, src_hbm, dst_hbm, d2_hbm, pos_v, src_v, dst_v, d2_v):
    cidx = lax.axis_index("c")
    sidx = lax.axis_index("s")
    wid = cidx * NSUB + sidx
    pltpu.sync_copy(pos_hbm, pos_v)
    pltpu.sync_copy(src_hbm.at[wid], src_v)
    pltpu.sync_copy(dst_hbm.at[wid], dst_v)

    def chunk(c, carry):
        for k in range(CHUNK // 16):
            sl = pl.ds(k * 16, 16)
            s = src_v[c, sl]
            d = dst_v[c, sl]
            s3 = s * 3
            d3 = d * 3
            dx = plsc.load_gather(pos_v, [d3]) - plsc.load_gather(pos_v, [s3])
            dy = plsc.load_gather(pos_v, [d3 + 1]) - plsc.load_gather(pos_v, [s3 + 1])
            dz = plsc.load_gather(pos_v, [d3 + 2]) - plsc.load_gather(pos_v, [s3 + 2])
            d2_v[c, sl] = dx * dx + dy * dy + dz * dz
        return carry

    lax.fori_loop(0, NCHUNK, chunk, 0)
    pltpu.sync_copy(d2_v, d2_hbm.at[wid])


def _make_edge_prep(interpret=False):
    return pl.kernel(
        _edge_prep_body,
        out_type=jax.ShapeDtypeStruct((NW, NCHUNK, CHUNK), jnp.float32),
        mesh=_mesh(),
        compiler_params=pltpu.CompilerParams(needs_layout_passes=False),
        scratch_types=[
            pltpu.VMEM((N * 3,), jnp.float32),
            pltpu.VMEM((NCHUNK, CHUNK), jnp.int32),
            pltpu.VMEM((NCHUNK, CHUNK), jnp.int32),
            pltpu.VMEM((NCHUNK, CHUNK), jnp.float32),
        ],
        interpret=interpret,
    )


def _fcut_x0_body(d2_ref, sp_ref, emb_ref, fcut_ref, x0_ref):
    d2 = d2_ref[...]
    dist = jnp.sqrt(d2 + 1e-9)
    r = jnp.clip(dist / CUTOFF, 0.0, 1.0)
    fcut_ref[...] = 0.5 * (jnp.cos(jnp.pi * r) + 1.0)
    sp = sp_ref[...]
    acc = jnp.zeros((sp.shape[0], D), jnp.float32)
    for s in range(NSPECIES):
        mask = (sp == s).astype(jnp.float32)
        acc = acc + mask * emb_ref[s, :][None, :]
    x0_ref[...] = acc


def _make_fcut_x0(interpret=False):
    return pl.pallas_call(
        _fcut_x0_body,
        out_shape=[
            jax.ShapeDtypeStruct((E // D, D), jnp.float32),
            jax.ShapeDtypeStruct((N, D), jnp.float32),
        ],
        interpret=interpret,
    )


def _spmm_body(x_hbm, src_hbm, dst_hbm, fcut_hbm, out_hbm,
               src_v, dst_v, fcut_v, rows_v, agg_sp, sem):
    cidx = lax.axis_index("c")
    sidx = lax.axis_index("s")
    wid = cidx * NSUB + sidx

    pltpu.sync_copy(src_hbm.at[wid], src_v)
    pltpu.sync_copy(dst_hbm.at[wid], dst_v)
    pltpu.sync_copy(fcut_hbm.at[wid], fcut_v)

    def zrow(r, carry):
        for k in range(D // 16):
            rows_v[r, pl.ds(k * 16, 16)] = jnp.zeros((16,), jnp.float32)
        return carry

    lax.fori_loop(0, CHUNK, zrow, 0)
    base = sidx * ROWS_PER_SUB
    for i in range(ROWS_PER_SUB // CHUNK):
        pltpu.sync_copy(rows_v, agg_sp.at[pl.ds(base + i * CHUNK, CHUNK)])
    rem = ROWS_PER_SUB % CHUNK
    if rem:
        pltpu.sync_copy(rows_v.at[pl.ds(0, rem)],
                        agg_sp.at[pl.ds(base + (ROWS_PER_SUB // CHUNK) * CHUNK, rem)])
    plsc.subcore_barrier()

    def chunk(j, carry):
        pltpu.async_copy(x_hbm.at[src_v.at[j]], rows_v, sem).wait()
        js = jnp.full((16,), j, dtype=jnp.int32)
        for r in range(CHUNK):
            f = plsc.load_gather(fcut_v, [js, jnp.full((16,), r, dtype=jnp.int32)])
            for k in range(D // 16):
                sl = pl.ds(k * 16, 16)
                rows_v[r, sl] = rows_v[r, sl] * f
        pltpu.sync_copy(rows_v, agg_sp.at[dst_v.at[j]], add=True)
        return carry

    lax.fori_loop(0, NCHUNK, chunk, 0)
    plsc.subcore_barrier()

    pltpu.sync_copy(agg_sp.at[pl.ds(sidx * ROWS_PER_SUB, ROWS_PER_SUB)],
                    out_hbm.at[cidx, pl.ds(sidx * ROWS_PER_SUB, ROWS_PER_SUB)])


def _make_spmm(interpret=False):
    return pl.kernel(
        _spmm_body,
        out_type=jax.ShapeDtypeStruct((2, N, D), jnp.float32),
        mesh=_mesh(),
        compiler_params=pltpu.CompilerParams(needs_layout_passes=False,
                                             use_tc_tiling_on_sc=False),
        scratch_types=[
            pltpu.VMEM((NCHUNK, CHUNK), jnp.int32),
            pltpu.VMEM((NCHUNK, CHUNK), jnp.int32),
            pltpu.VMEM((NCHUNK, CHUNK), jnp.float32),
            pltpu.VMEM((CHUNK, D), jnp.float32),
            pltpu.VMEM_SHARED((N, D), jnp.float32),
            pltpu.SemaphoreType.DMA,
        ],
        interpret=interpret,
    )


def _mix_body(x_ref, p0_ref, p1_ref, w_ref, out_ref):
    agg = p0_ref[...] + p1_ref[...]
    x = x_ref[...]
    t = jnp.tanh(jnp.dot(agg, w_ref[...], preferred_element_type=jnp.float32))
    out_ref[...] = x + t * x


def _make_mix(interpret=False):
    return pl.pallas_call(
        _mix_body,
        out_shape=jax.ShapeDtypeStruct((N, D), jnp.float32),
        interpret=interpret,
    )


def _final_body(x_ref, p0_ref, p1_ref, w_ref, wout_ref, sp_ref, comp_ref, out_ref):
    agg = p0_ref[...] + p1_ref[...]
    x = x_ref[...]
    t = jnp.tanh(jnp.dot(agg, w_ref[...], preferred_element_type=jnp.float32))
    x2 = x + t * x
    e = jnp.dot(x2, wout_ref[...], preferred_element_type=jnp.float32)
    onehot = (sp_ref[...] == lax.broadcasted_iota(jnp.int32, (1, NSPECIES), 1))
    comp_e = jnp.sum(jnp.where(onehot, comp_ref[...], 0.0))
    out_ref[0, 0] = jnp.sum(e) + comp_e


def _make_final(interpret=False):
    return pl.pallas_call(
        _final_body,
        out_shape=jax.ShapeDtypeStruct((1, 1), jnp.float32),
        out_specs=pl.BlockSpec(memory_space=pltpu.SMEM),
        interpret=interpret,
    )


def _run(positions, species, edge_index, embeddings, W_msg, W_out,
         composition_weights, interpret=False):
    src = edge_index[0].astype(jnp.int32).reshape(NW, NCHUNK, CHUNK)
    dst = edge_index[1].astype(jnp.int32).reshape(NW, NCHUNK, CHUNK)
    sp32 = species.astype(jnp.int32).reshape(N, 1)
    pos_flat = positions.astype(jnp.float32).reshape(N * 3)

    d2 = _make_edge_prep(interpret)(pos_flat, src, dst)
    fcut2d, x0 = _make_fcut_x0(interpret)(d2.reshape(E // D, D), sp32, embeddings)
    fcut = fcut2d.reshape(NW, NCHUNK, CHUNK)

    spmm = _make_spmm(interpret)
    mix = _make_mix(interpret)

    x = x0
    for l in range(NLAYERS - 1):
        partials = spmm(x, src, dst, fcut)
        x = mix(x, partials[0], partials[1], W_msg[l])
    partials = spmm(x, src, dst, fcut)
    out = _make_final(interpret)(x, partials[0], partials[1],
                                 W_msg[NLAYERS - 1], W_out, sp32,
                                 composition_weights.reshape(1, NSPECIES))
    return out.reshape(())


def kernel(positions, species, edge_index, embeddings, W_msg, W_out,
           composition_weights):
    return _run(positions, species, edge_index, embeddings, W_msg, W_out,
                composition_weights)


# speedup vs baseline: 10.6869x; 1.3940x over previous
import jax
import jax.numpy as jnp
from jax import lax
from jax.experimental import pallas as pl
from jax.experimental.pallas import tpu as pltpu
from jax.experimental.pallas import tpu_sc as plsc

N = 10000
E = 320000
D = 128
NSPECIES = 8
CUTOFF = 5.0
NLAYERS = 2

NW = 32
NSUB = 16
EPW = E // NW
CHUNK = 80
NCHUNK = EPW // CHUNK
ROWS_PER_SUB = N // NSUB

_mesh = lambda: plsc.VectorSubcoreMesh(core_axis_name="c", subcore_axis_name="s",
                                       num_cores=2, num_subcores=16)


def _edge_prep_body(pos_hbm, src_hbm, dst_hbm, d2_hbm, pos_v, src_v, dst_v, d2_v):
    cidx = lax.axis_index("c")
    sidx = lax.axis_index("s")
    wid = cidx * NSUB + sidx
    pltpu.sync_copy(pos_hbm, pos_v)
    pltpu.sync_copy(src_hbm.at[wid], src_v)
    pltpu.sync_copy(dst_hbm.at[wid], dst_v)

    def chunk(c, carry):
        for k in range(CHUNK // 16):
            sl = pl.ds(k * 16, 16)
            s = src_v[c, sl]
            d = dst_v[c, sl]
            s3 = s * 3
            d3 = d * 3
            dx = plsc.load_gather(pos_v, [d3]) - plsc.load_gather(pos_v, [s3])
            dy = plsc.load_gather(pos_v, [d3 + 1]) - plsc.load_gather(pos_v, [s3 + 1])
            dz = plsc.load_gather(pos_v, [d3 + 2]) - plsc.load_gather(pos_v, [s3 + 2])
            d2_v[c, sl] = dx * dx + dy * dy + dz * dz
        return carry

    lax.fori_loop(0, NCHUNK, chunk, 0)
    pltpu.sync_copy(d2_v, d2_hbm.at[wid])


def _make_edge_prep(interpret=False):
    return pl.kernel(
        _edge_prep_body,
        out_type=jax.ShapeDtypeStruct((NW, NCHUNK, CHUNK), jnp.float32),
        mesh=_mesh(),
        compiler_params=pltpu.CompilerParams(needs_layout_passes=False),
        scratch_types=[
            pltpu.VMEM((N * 3,), jnp.float32),
            pltpu.VMEM((NCHUNK, CHUNK), jnp.int32),
            pltpu.VMEM((NCHUNK, CHUNK), jnp.int32),
            pltpu.VMEM((NCHUNK, CHUNK), jnp.float32),
        ],
        interpret=interpret,
    )


def _fcut_x0_body(d2_ref, sp_ref, emb_ref, fcut_ref, x0_ref):
    d2 = d2_ref[...]
    dist = jnp.sqrt(d2 + 1e-9)
    r = jnp.clip(dist / CUTOFF, 0.0, 1.0)
    fcut_ref[...] = 0.5 * (jnp.cos(jnp.pi * r) + 1.0)
    sp = sp_ref[...]
    acc = jnp.zeros((sp.shape[0], D), jnp.float32)
    for s in range(NSPECIES):
        mask = (sp == s).astype(jnp.float32)
        acc = acc + mask * emb_ref[s, :][None, :]
    x0_ref[...] = acc


def _make_fcut_x0(interpret=False):
    return pl.pallas_call(
        _fcut_x0_body,
        out_shape=[
            jax.ShapeDtypeStruct((E // D, D), jnp.float32),
            jax.ShapeDtypeStruct((N, D), jnp.float32),
        ],
        interpret=interpret,
    )


KCH = 400
KNC = EPW // KCH
NSP16 = 16


def _ksum_body(sp_hbm, src_hbm, dst_hbm, fcut_hbm, out_hbm,
               sp_v, src_v, dst_v, fcut_v, srows_v, agg_sp):
    cidx = lax.axis_index("c")
    sidx = lax.axis_index("s")
    wid = cidx * NSUB + sidx
    pltpu.sync_copy(sp_hbm, sp_v)
    pltpu.sync_copy(src_hbm.at[wid], src_v)
    pltpu.sync_copy(dst_hbm.at[wid], dst_v)
    pltpu.sync_copy(fcut_hbm.at[wid], fcut_v)

    def zrow(r, carry):
        srows_v[r, :] = jnp.zeros((16,), jnp.float32)
        return carry

    lax.fori_loop(0, KCH, zrow, 0)
    base = sidx * ROWS_PER_SUB
    pltpu.sync_copy(srows_v, agg_sp.at[pl.ds(base, KCH)])
    pltpu.sync_copy(srows_v.at[pl.ds(0, ROWS_PER_SUB - KCH)],
                    agg_sp.at[pl.ds(base + KCH, ROWS_PER_SUB - KCH)])
    plsc.subcore_barrier()

    iota = lax.broadcasted_iota(jnp.int32, (16,), 0)

    def chunk(j, carry):
        lax.fori_loop(0, KCH, zrow, 0)
        for k in range(KCH // 16):
            sl = pl.ds(k * 16, 16)
            s16 = src_v[j, sl]
            spv = plsc.load_gather(sp_v, [s16])
            f16 = fcut_v[j, sl]
            plsc.store_scatter(srows_v, [iota + k * 16, spv], f16)
        pltpu.sync_copy(srows_v, agg_sp.at[dst_v.at[j]], add=True)
        return carry

    lax.fori_loop(0, KNC, chunk, 0)
    plsc.subcore_barrier()
    pltpu.sync_copy(agg_sp.at[pl.ds(sidx * ROWS_PER_SUB, ROWS_PER_SUB)],
                    out_hbm.at[cidx, pl.ds(sidx * ROWS_PER_SUB, ROWS_PER_SUB)])


def _make_ksum(interpret=False):
    return pl.kernel(
        _ksum_body,
        out_type=jax.ShapeDtypeStruct((2, N, NSP16), jnp.float32),
        mesh=_mesh(),
        compiler_params=pltpu.CompilerParams(needs_layout_passes=False,
                                             use_tc_tiling_on_sc=False),
        scratch_types=[
            pltpu.VMEM((N,), jnp.int32),
            pltpu.VMEM((KNC, KCH), jnp.int32),
            pltpu.VMEM((KNC, KCH), jnp.int32),
            pltpu.VMEM((KNC, KCH), jnp.float32),
            pltpu.VMEM((KCH, NSP16), jnp.float32),
            pltpu.VMEM_SHARED((N, NSP16), jnp.float32),
        ],
        interpret=interpret,
    )


def _mix0_body(x_ref, s0_ref, s1_ref, emb_ref, w_ref, out_ref):
    S = s0_ref[...] + s1_ref[...]
    agg = jnp.zeros(x_ref.shape, jnp.float32)
    for s in range(NSPECIES):
        agg = agg + S[:, s][:, None] * emb_ref[s, :][None, :]
    x = x_ref[...]
    t = jnp.tanh(jnp.dot(agg, w_ref[...], preferred_element_type=jnp.float32))
    out_ref[...] = x + t * x


def _make_mix0(interpret=False):
    return pl.pallas_call(
        _mix0_body,
        out_shape=jax.ShapeDtypeStruct((N, D), jnp.float32),
        interpret=interpret,
    )


def _spmm_body(x_hbm, src_hbm, dst_hbm, fcut_hbm, out_hbm,
               src_v, dst_v, fcut_v, rows_v, agg_sp, sem):
    cidx = lax.axis_index("c")
    sidx = lax.axis_index("s")
    wid = cidx * NSUB + sidx

    pltpu.sync_copy(src_hbm.at[wid], src_v)
    pltpu.sync_copy(dst_hbm.at[wid], dst_v)
    pltpu.sync_copy(fcut_hbm.at[wid], fcut_v)

    def zrow(r, carry):
        for k in range(D // 16):
            rows_v[r, pl.ds(k * 16, 16)] = jnp.zeros((16,), jnp.float32)
        return carry

    lax.fori_loop(0, CHUNK, zrow, 0)
    base = sidx * ROWS_PER_SUB
    for i in range(ROWS_PER_SUB // CHUNK):
        pltpu.sync_copy(rows_v, agg_sp.at[pl.ds(base + i * CHUNK, CHUNK)])
    rem = ROWS_PER_SUB % CHUNK
    if rem:
        pltpu.sync_copy(rows_v.at[pl.ds(0, rem)],
                        agg_sp.at[pl.ds(base + (ROWS_PER_SUB // CHUNK) * CHUNK, rem)])
    plsc.subcore_barrier()

    def chunk(j, carry):
        pltpu.async_copy(x_hbm.at[src_v.at[j]], rows_v, sem).wait()
        js = jnp.full((16,), j, dtype=jnp.int32)
        for r in range(CHUNK):
            f = plsc.load_gather(fcut_v, [js, jnp.full((16,), r, dtype=jnp.int32)])
            for k in range(D // 16):
                sl = pl.ds(k * 16, 16)
                rows_v[r, sl] = rows_v[r, sl] * f
        pltpu.sync_copy(rows_v, agg_sp.at[dst_v.at[j]], add=True)
        return carry

    lax.fori_loop(0, NCHUNK, chunk, 0)
    plsc.subcore_barrier()

    pltpu.sync_copy(agg_sp.at[pl.ds(sidx * ROWS_PER_SUB, ROWS_PER_SUB)],
                    out_hbm.at[cidx, pl.ds(sidx * ROWS_PER_SUB, ROWS_PER_SUB)])


def _make_spmm(interpret=False):
    return pl.kernel(
        _spmm_body,
        out_type=jax.ShapeDtypeStruct((2, N, D), jnp.float32),
        mesh=_mesh(),
        compiler_params=pltpu.CompilerParams(needs_layout_passes=False,
                                             use_tc_tiling_on_sc=False),
        scratch_types=[
            pltpu.VMEM((NCHUNK, CHUNK), jnp.int32),
            pltpu.VMEM((NCHUNK, CHUNK), jnp.int32),
            pltpu.VMEM((NCHUNK, CHUNK), jnp.float32),
            pltpu.VMEM((CHUNK, D), jnp.float32),
            pltpu.VMEM_SHARED((N, D), jnp.float32),
            pltpu.SemaphoreType.DMA,
        ],
        interpret=interpret,
    )


def _mix_body(x_ref, p0_ref, p1_ref, w_ref, out_ref):
    agg = p0_ref[...] + p1_ref[...]
    x = x_ref[...]
    t = jnp.tanh(jnp.dot(agg, w_ref[...], preferred_element_type=jnp.float32))
    out_ref[...] = x + t * x


def _make_mix(interpret=False):
    return pl.pallas_call(
        _mix_body,
        out_shape=jax.ShapeDtypeStruct((N, D), jnp.float32),
        interpret=interpret,
    )


def _final_body(x_ref, p0_ref, p1_ref, w_ref, wout_ref, sp_ref, comp_ref, out_ref):
    agg = p0_ref[...] + p1_ref[...]
    x = x_ref[...]
    t = jnp.tanh(jnp.dot(agg, w_ref[...], preferred_element_type=jnp.float32))
    x2 = x + t * x
    e = jnp.dot(x2, wout_ref[...], preferred_element_type=jnp.float32)
    onehot = (sp_ref[...] == lax.broadcasted_iota(jnp.int32, (1, NSPECIES), 1))
    comp_e = jnp.sum(jnp.where(onehot, comp_ref[...], 0.0))
    out_ref[0, 0] = jnp.sum(e) + comp_e


def _make_final(interpret=False):
    return pl.pallas_call(
        _final_body,
        out_shape=jax.ShapeDtypeStruct((1, 1), jnp.float32),
        out_specs=pl.BlockSpec(memory_space=pltpu.SMEM),
        interpret=interpret,
    )


def _run(positions, species, edge_index, embeddings, W_msg, W_out,
         composition_weights, interpret=False):
    src = edge_index[0].astype(jnp.int32).reshape(NW, NCHUNK, CHUNK)
    dst = edge_index[1].astype(jnp.int32).reshape(NW, NCHUNK, CHUNK)
    sp32 = species.astype(jnp.int32).reshape(N, 1)
    pos_flat = positions.astype(jnp.float32).reshape(N * 3)

    d2 = _make_edge_prep(interpret)(pos_flat, src, dst)
    fcut2d, x0 = _make_fcut_x0(interpret)(d2.reshape(E // D, D), sp32, embeddings)
    fcut = fcut2d.reshape(NW, NCHUNK, CHUNK)

    spmm = _make_spmm(interpret)
    mix = _make_mix(interpret)

    spK = species.astype(jnp.int32)
    srcK = src.reshape(NW, KNC, KCH)
    dstK = dst.reshape(NW, KNC, KCH)
    fcutK = fcut2d.reshape(NW, KNC, KCH)
    S = _make_ksum(interpret)(spK, srcK, dstK, fcutK)
    x = _make_mix0(interpret)(x0, S[0], S[1], embeddings, W_msg[0])

    for l in range(1, NLAYERS - 1):
        partials = spmm(x, src, dst, fcut)
        x = mix(x, partials[0], partials[1], W_msg[l])
    partials = spmm(x, src, dst, fcut)
    out = _make_final(interpret)(x, partials[0], partials[1],
                                 W_msg[NLAYERS - 1], W_out, sp32,
                                 composition_weights.reshape(1, NSPECIES))
    return out.reshape(())


def kernel(positions, species, edge_index, embeddings, W_msg, W_out,
           composition_weights):
    return _run(positions, species, edge_index, embeddings, W_msg, W_out,
                composition_weights)
